# baseline (speedup 1.0000x reference)
import functools
import jax
import jax.numpy as jnp
from jax import lax
from jax.experimental import pallas as pl
from jax.experimental.pallas import tpu as pltpu
from jax.experimental.pallas import tpu_sc as plsc

N = 2048
IN_CH = 128
HID = 64
HEADS = 4
OUT_CH = 128

M_TOTAL = N * N
NUM_EDGES = M_TOTAL - 1 - (9 * (M_TOTAL - 1)) // 10

ROWS_A = 64
ROWS_C = 256
BD = 256
NPIV = 16


def _count_lanes(blk, piv_ref):
    lane = lax.broadcasted_iota(jnp.int32, (1, 128), 1)
    acc = jnp.zeros((1, 128), jnp.float32)
    for p in range(NPIV):
        cnt = jnp.sum((blk >= piv_ref[0, p]).astype(jnp.float32))
        acc = acc + jnp.where(lane == p, cnt, 0.0)
    return acc


def _agg_body(attn_ref, w_ref, b_ref, piv_ref, out_ref, cnt_ref):
    def rnd(v):
        return v.astype(jnp.bfloat16).astype(jnp.float32)

    acc = rnd(w_ref[0, 0]) * rnd(attn_ref[0, 0])
    for c in range(1, 12):
        acc = acc + rnd(w_ref[0, c]) * rnd(attn_ref[0, c])
    agg = acc + b_ref[0, 0]
    out_ref[...] = agg

    @pl.when(pl.program_id(0) == 0)
    def _init():
        cnt_ref[...] = jnp.zeros_like(cnt_ref)

    cnt_ref[...] += _count_lanes(agg, piv_ref)


def _agg_mat(attn, agg_w, agg_b, pivots):
    piv = jnp.full((1, 128), jnp.inf, jnp.float32).at[0, :NPIV].set(pivots)
    return pl.pallas_call(
        _agg_body,
        grid=(N // ROWS_A,),
        in_specs=[
            pl.BlockSpec((1, 12, ROWS_A, N), lambda i: (0, 0, i, 0)),
            pl.BlockSpec(memory_space=pltpu.SMEM),
            pl.BlockSpec(memory_space=pltpu.SMEM),
            pl.BlockSpec(memory_space=pltpu.SMEM),
        ],
        out_specs=[
            pl.BlockSpec((ROWS_A, N), lambda i: (i, 0)),
            pl.BlockSpec((1, 128), lambda i: (0, 0)),
        ],
        out_shape=[
            jax.ShapeDtypeStruct((N, N), jnp.float32),
            jax.ShapeDtypeStruct((1, 128), jnp.float32),
        ],
    )(attn, agg_w.reshape(1, 12), agg_b.reshape(1, 1), piv)


def _count_body(agg_ref, piv_ref, out_ref):
    @pl.when(pl.program_id(0) == 0)
    def _init():
        out_ref[...] = jnp.zeros_like(out_ref)

    out_ref[...] += _count_lanes(agg_ref[...], piv_ref)


def _count_ge(agg, pivots):
    piv = jnp.full((1, 128), jnp.inf, jnp.float32).at[0, :NPIV].set(pivots)
    out = pl.pallas_call(
        _count_body,
        grid=(N // ROWS_C,),
        in_specs=[
            pl.BlockSpec((ROWS_C, N), lambda i: (i, 0)),
            pl.BlockSpec(memory_space=pltpu.SMEM),
        ],
        out_specs=pl.BlockSpec((1, 128), lambda i: (0, 0)),
        out_shape=jax.ShapeDtypeStruct((1, 128), jnp.float32),
    )(agg, piv)
    return out[0, :NPIV]


NBINS_SC = 4096
SC_NW = 32
SC_CHUNK = 32768
SC_ELEMS = M_TOTAL // SC_NW


def _sc_hist_body(agg_ref, lo_ref, sc_ref, zero_ref, out_ref,
                  chunk, hist, lovec, scvec):
    wid = lax.axis_index("s") * 2 + lax.axis_index("c")
    base = wid * SC_ELEMS
    pltpu.sync_copy(lo_ref, lovec)
    pltpu.sync_copy(sc_ref, scvec)
    pltpu.sync_copy(zero_ref, hist)
    lo_v = lovec[...]
    sc_v = scvec[...]
    lane = lax.iota(jnp.int32, 16)
    ones = jnp.full((16,), 1.0, jnp.float32)
    for c in range(SC_ELEMS // SC_CHUNK):
        pltpu.sync_copy(agg_ref.at[pl.ds(base + c * SC_CHUNK, SC_CHUNK)],
                        chunk)

        def body(i, carry):
            for u in range(4):
                v = chunk[pl.ds(i * 64 + u * 16, 16)]
                t = (v - lo_v) * sc_v + 131072.0
                idx = jnp.clip(t.astype(jnp.int32) - 131072, 0, NBINS_SC - 1)
                plsc.addupdate_scatter(hist, [idx * 16 + lane], ones)
            return carry

        lax.fori_loop(0, SC_CHUNK // 64, body, 0)
    pltpu.sync_copy(hist, out_ref.at[wid])


def _sc_hist(agg1d, lo, hi):
    scale = jnp.float32(NBINS_SC) / (hi - lo)
    lo16 = jnp.full((16,), lo, jnp.float32)
    sc16 = jnp.full((16,), scale, jnp.float32)
    zeros = jnp.zeros((NBINS_SC * 16,), jnp.float32)
    mesh = plsc.VectorSubcoreMesh(core_axis_name="c", subcore_axis_name="s")
    f = pl.kernel(
        _sc_hist_body,
        out_type=jax.ShapeDtypeStruct((SC_NW, NBINS_SC * 16), jnp.float32),
        mesh=mesh,
        compiler_params=pltpu.CompilerParams(needs_layout_passes=False),
        scratch_types=[
            pltpu.VMEM((SC_CHUNK,), jnp.float32),
            pltpu.VMEM((NBINS_SC * 16,), jnp.float32),
            pltpu.VMEM((16,), jnp.float32),
            pltpu.VMEM((16,), jnp.float32),
        ],
    )
    out = f(agg1d, lo16, sc16, zeros)
    return jnp.sum(out.reshape(SC_NW, NBINS_SC, 16), axis=(0, 2))


def _row_count_body(agg_ref, t_ref, out_ref):
    msk = (agg_ref[...] >= t_ref[0, 0]).astype(jnp.float32)
    out_ref[...] = jnp.sum(msk, axis=1, keepdims=True)


def _row_counts(agg, thresh):
    return pl.pallas_call(
        _row_count_body,
        grid=(N // ROWS_C,),
        in_specs=[
            pl.BlockSpec((ROWS_C, N), lambda i: (i, 0)),
            pl.BlockSpec(memory_space=pltpu.SMEM),
        ],
        out_specs=pl.BlockSpec((ROWS_C, 1), lambda i: (i, 0)),
        out_shape=jax.ShapeDtypeStruct((N, 1), jnp.float32),
    )(agg, thresh.reshape(1, 1))


def _gat_body(agg_ref, x_ref, xd_ref, w_ref, atts_ref, attd_ref, mode_ref,
              allow_ref, t_ref, bias_ref, out_ref, *, heads, out_ch, do_elu):
    j = pl.program_id(0)
    dst0 = j * BD
    agg = agg_ref[...]
    mode = mode_ref[...]
    allow = allow_ref[0:1, pl.ds(dst0, BD)]
    msk = (agg >= t_ref[0, 0]) & (
        (mode == 1.0) | ((mode == 2.0) & (allow == 1.0)))

    rows = lax.broadcasted_iota(jnp.int32, (N, BD), 0)
    cols = lax.broadcasted_iota(jnp.int32, (N, BD), 1)
    diag = rows == (cols + dst0)

    xl = jnp.dot(x_ref[...], w_ref[...], preferred_element_type=jnp.float32)
    xld = jnp.dot(xd_ref[...], w_ref[...], preferred_element_type=jnp.float32)

    outs = []
    for h in range(heads):
        xh = xl[:, h * out_ch:(h + 1) * out_ch]
        xhd = xld[:, h * out_ch:(h + 1) * out_ch]
        a_s = jnp.sum(xh * atts_ref[h:h + 1, :], axis=1, keepdims=True)
        a_d = lax.dot_general(attd_ref[h:h + 1, :], xhd,
                              (((1,), (1,)), ((), ())),
                              preferred_element_type=jnp.float32,
                              precision=lax.Precision.HIGHEST)
        logit = a_s + a_d
        logit = jnp.where(logit >= 0.0, logit, 0.2 * logit)
        e = jnp.exp(logit)
        w = jnp.where(msk, e, 0.0) + jnp.where(diag, e, 0.0)
        denom = jnp.sum(w, axis=0, keepdims=True) + 1e-16
        w = w / denom
        w_hi = w.astype(jnp.bfloat16).astype(jnp.float32)
        w_lo = w - w_hi
        x_hi = xh.astype(jnp.bfloat16).astype(jnp.float32)
        x_lo = xh - x_hi
        dims = (((0,), (0,)), ((), ()))
        o = (lax.dot_general(w_hi, x_hi, dims,
                             preferred_element_type=jnp.float32)
             + lax.dot_general(w_hi, x_lo, dims,
                               preferred_element_type=jnp.float32)
             + lax.dot_general(w_lo, x_hi, dims,
                               preferred_element_type=jnp.float32))
        outs.append(o)

    out = outs[0] if heads == 1 else jnp.concatenate(outs, axis=1)
    out = out + bias_ref[0:1, :]
    if do_elu:
        out = jnp.where(out > 0.0, out, jnp.exp(out) - 1.0)
    out_ref[...] = out


def _gat_layer(agg, x, W, att_src, att_dst, mode, allow, thresh, bias,
               heads, out_ch, do_elu):
    body = functools.partial(_gat_body, heads=heads, out_ch=out_ch,
                             do_elu=do_elu)
    feat = heads * out_ch
    in_ch = x.shape[1]
    return pl.pallas_call(
        body,
        grid=(N // BD,),
        in_specs=[
            pl.BlockSpec((N, BD), lambda j: (0, j)),
            pl.BlockSpec((N, in_ch), lambda j: (0, 0)),
            pl.BlockSpec((BD, in_ch), lambda j: (j, 0)),
            pl.BlockSpec((in_ch, feat), lambda j: (0, 0)),
            pl.BlockSpec((heads, out_ch), lambda j: (0, 0)),
            pl.BlockSpec((heads, out_ch), lambda j: (0, 0)),
            pl.BlockSpec((N, 1), lambda j: (0, 0)),
            pl.BlockSpec((1, N), lambda j: (0, 0)),
            pl.BlockSpec(memory_space=pltpu.SMEM),
            pl.BlockSpec((1, feat), lambda j: (0, 0)),
        ],
        out_specs=pl.BlockSpec((BD, feat), lambda j: (j, 0)),
        out_shape=jax.ShapeDtypeStruct((N, feat), jnp.float32),
    )(agg, x, x, W, att_src.reshape(heads, out_ch),
      att_dst.reshape(heads, out_ch), mode, allow, thresh.reshape(1, 1),
      bias.reshape(1, feat))


def _bisect_step(lo, hi, cnts, pivots, target):
    sel = jnp.sum((cnts >= target).astype(jnp.int32))
    piv_ext = jnp.concatenate([pivots, hi[None]])
    lo = piv_ext[sel - 1]
    hi = piv_ext[sel]
    return lo, hi, sel


def _pivots(lo, hi):
    return lo + (hi - lo) / NPIV * jnp.arange(NPIV, dtype=jnp.float32)


def kernel(x, attn_tensor, agg_w, agg_b, W1, att_src1, att_dst1, b1,
           W2, att_src2, att_dst2, b2):
    target = jnp.float32(NUM_EDGES)
    lo = jnp.sum(jnp.minimum(agg_w, 0.0)) + agg_b - 1e-3
    hi = jnp.sum(jnp.maximum(agg_w, 0.0)) + agg_b + 1e-3

    piv1 = _pivots(lo, hi)
    agg, cnt1 = _agg_mat(attn_tensor, agg_w, agg_b, piv1)
    lo, hi, _ = _bisect_step(lo, hi, cnt1[0, :NPIV], piv1, target)

    pivots = _pivots(lo, hi)
    cnts = _count_ge(agg, pivots)
    lo, hi, _ = _bisect_step(lo, hi, cnts, pivots, target)

    hist = _sc_hist(agg.reshape(-1), lo, hi)
    sfx = jnp.cumsum(hist[::-1])[::-1]
    b = jnp.sum((sfx >= target).astype(jnp.int32)) - 1
    thresh = lo + b.astype(jnp.float32) * (hi - lo) / NBINS_SC
    rcnt = _row_counts(agg, thresh)[:, 0]

    excl = jnp.cumsum(rcnt) - rcnt
    limit = jnp.clip(target - excl, 0.0, rcnt)
    mode = jnp.where(limit >= rcnt, 1.0,
                     jnp.where(limit > 0.0, 2.0, 0.0))
    is_part = mode == 2.0
    rstar = jnp.argmax(is_part)
    rowvals = lax.dynamic_slice(agg, (rstar, 0), (1, N))[0]
    inrow = (rowvals >= thresh).astype(jnp.float32)
    pfx = jnp.cumsum(inrow) - inrow
    lim_r = limit[rstar]
    allow = jnp.where(jnp.any(is_part), (pfx < lim_r).astype(jnp.float32),
                      jnp.ones((N,), jnp.float32))
    mode = mode.reshape(N, 1)
    allow = allow.reshape(1, N)

    h1 = _gat_layer(agg, x, W1, att_src1, att_dst1, mode, allow, thresh, b1,
                    HEADS, HID, do_elu=True)

    out = _gat_layer(agg, h1, W2, att_src2, att_dst2, mode, allow, thresh, b2,
                     1, OUT_CH, do_elu=False)
    return out


# speedup vs baseline: 1.1364x; 1.1364x over previous
import functools
import jax
import jax.numpy as jnp
from jax import lax
from jax.experimental import pallas as pl
from jax.experimental.pallas import tpu as pltpu
from jax.experimental.pallas import tpu_sc as plsc

N = 2048
IN_CH = 128
HID = 64
HEADS = 4
OUT_CH = 128

M_TOTAL = N * N
NUM_EDGES = M_TOTAL - 1 - (9 * (M_TOTAL - 1)) // 10

ROWS_A = 64
ROWS_C = 256
BD = 256
NPIV = 16


def _count_lanes(blk, piv_ref):
    lane = lax.broadcasted_iota(jnp.int32, (1, 128), 1)
    acc = jnp.zeros((1, 128), jnp.float32)
    for p in range(NPIV):
        cnt = jnp.sum((blk >= piv_ref[0, p]).astype(jnp.float32))
        acc = acc + jnp.where(lane == p, cnt, 0.0)
    return acc


def _agg_body(attn_ref, w_ref, b_ref, piv_ref, out_ref, cnt_ref):
    def rnd(v):
        return v.astype(jnp.bfloat16).astype(jnp.float32)

    acc = rnd(w_ref[0, 0]) * rnd(attn_ref[0, 0])
    for c in range(1, 12):
        acc = acc + rnd(w_ref[0, c]) * rnd(attn_ref[0, c])
    agg = acc + b_ref[0, 0]
    out_ref[...] = agg

    @pl.when(pl.program_id(0) == 0)
    def _init():
        cnt_ref[...] = jnp.zeros_like(cnt_ref)

    cnt_ref[...] += _count_lanes(agg, piv_ref)


def _agg_mat(attn, agg_w, agg_b, pivots):
    piv = jnp.full((1, 128), jnp.inf, jnp.float32).at[0, :NPIV].set(pivots)
    return pl.pallas_call(
        _agg_body,
        grid=(N // ROWS_A,),
        in_specs=[
            pl.BlockSpec((1, 12, ROWS_A, N), lambda i: (0, 0, i, 0)),
            pl.BlockSpec(memory_space=pltpu.SMEM),
            pl.BlockSpec(memory_space=pltpu.SMEM),
            pl.BlockSpec(memory_space=pltpu.SMEM),
        ],
        out_specs=[
            pl.BlockSpec((ROWS_A, N), lambda i: (i, 0)),
            pl.BlockSpec((1, 128), lambda i: (0, 0)),
        ],
        out_shape=[
            jax.ShapeDtypeStruct((N, N), jnp.float32),
            jax.ShapeDtypeStruct((1, 128), jnp.float32),
        ],
    )(attn, agg_w.reshape(1, 12), agg_b.reshape(1, 1), piv)


def _count_body(agg_ref, piv_ref, out_ref):
    @pl.when(pl.program_id(0) == 0)
    def _init():
        out_ref[...] = jnp.zeros_like(out_ref)

    out_ref[...] += _count_lanes(agg_ref[...], piv_ref)


def _count_ge(agg, pivots):
    piv = jnp.full((1, 128), jnp.inf, jnp.float32).at[0, :NPIV].set(pivots)
    out = pl.pallas_call(
        _count_body,
        grid=(N // ROWS_C,),
        in_specs=[
            pl.BlockSpec((ROWS_C, N), lambda i: (i, 0)),
            pl.BlockSpec(memory_space=pltpu.SMEM),
        ],
        out_specs=pl.BlockSpec((1, 128), lambda i: (0, 0)),
        out_shape=jax.ShapeDtypeStruct((1, 128), jnp.float32),
    )(agg, piv)
    return out[0, :NPIV]


NBINS_SC = 4096
SC_NW = 32
SC_CHUNK = 32768
SC_ELEMS = M_TOTAL // SC_NW


def _sc_hist_body(agg_ref, sc_ref, c_ref, zero_ref, out_ref,
                  chunk, hist, scvec, cvec):
    wid = lax.axis_index("s") * 2 + lax.axis_index("c")
    base = wid * SC_ELEMS
    pltpu.sync_copy(sc_ref, scvec)
    pltpu.sync_copy(c_ref, cvec)
    pltpu.sync_copy(zero_ref, hist)
    sc_v = scvec[...]
    c_v = cvec[...]
    lane2 = lax.iota(jnp.int32, 16) - 16 * 131072
    ones = jnp.full((16,), 1.0, jnp.float32)
    for c in range(SC_ELEMS // SC_CHUNK):
        pltpu.sync_copy(agg_ref.at[pl.ds(base + c * SC_CHUNK, SC_CHUNK)],
                        chunk)

        def body(i, carry):
            for u in range(8):
                v = chunk[pl.ds(i * 128 + u * 16, 16)]
                t = v * sc_v + c_v
                t = jnp.minimum(jnp.maximum(t, 131072.0),
                                131072.0 + (NBINS_SC - 1))
                idx = t.astype(jnp.int32) * 16 + lane2
                plsc.addupdate_scatter(hist, [idx], ones)
            return carry

        lax.fori_loop(0, SC_CHUNK // 128, body, 0)
    pltpu.sync_copy(hist, out_ref.at[wid])


def _sc_hist(agg1d, lo, hi):
    scale = jnp.float32(NBINS_SC) / (hi - lo)
    sc16 = jnp.full((16,), scale, jnp.float32)
    c16 = jnp.full((16,), 131072.0 - lo * scale, jnp.float32)
    zeros = jnp.zeros((NBINS_SC * 16,), jnp.float32)
    mesh = plsc.VectorSubcoreMesh(core_axis_name="c", subcore_axis_name="s")
    f = pl.kernel(
        _sc_hist_body,
        out_type=jax.ShapeDtypeStruct((SC_NW, NBINS_SC * 16), jnp.float32),
        mesh=mesh,
        compiler_params=pltpu.CompilerParams(needs_layout_passes=False),
        scratch_types=[
            pltpu.VMEM((SC_CHUNK,), jnp.float32),
            pltpu.VMEM((NBINS_SC * 16,), jnp.float32),
            pltpu.VMEM((16,), jnp.float32),
            pltpu.VMEM((16,), jnp.float32),
        ],
    )
    out = f(agg1d, sc16, c16, zeros)
    return jnp.sum(out.reshape(SC_NW, NBINS_SC, 16), axis=(0, 2))


def _row_count_body(agg_ref, t_ref, out_ref):
    msk = (agg_ref[...] >= t_ref[0, 0]).astype(jnp.float32)
    out_ref[...] = jnp.sum(msk, axis=1, keepdims=True)


def _row_counts(agg, thresh):
    return pl.pallas_call(
        _row_count_body,
        grid=(N // ROWS_C,),
        in_specs=[
            pl.BlockSpec((ROWS_C, N), lambda i: (i, 0)),
            pl.BlockSpec(memory_space=pltpu.SMEM),
        ],
        out_specs=pl.BlockSpec((ROWS_C, 1), lambda i: (i, 0)),
        out_shape=jax.ShapeDtypeStruct((N, 1), jnp.float32),
    )(agg, thresh.reshape(1, 1))


def _gat_body(agg_ref, x_ref, xd_ref, w_ref, atts_ref, attd_ref, mode_ref,
              allow_ref, t_ref, bias_ref, out_ref, *, heads, out_ch, do_elu):
    j = pl.program_id(0)
    dst0 = j * BD
    agg = agg_ref[...]
    mode = mode_ref[...]
    allow = allow_ref[0:1, pl.ds(dst0, BD)]
    msk = (agg >= t_ref[0, 0]) & (
        (mode == 1.0) | ((mode == 2.0) & (allow == 1.0)))

    rows = lax.broadcasted_iota(jnp.int32, (N, BD), 0)
    cols = lax.broadcasted_iota(jnp.int32, (N, BD), 1)
    diag = rows == (cols + dst0)

    xl = jnp.dot(x_ref[...], w_ref[...], preferred_element_type=jnp.float32)
    xld = jnp.dot(xd_ref[...], w_ref[...], preferred_element_type=jnp.float32)

    outs = []
    for h in range(heads):
        xh = xl[:, h * out_ch:(h + 1) * out_ch]
        xhd = xld[:, h * out_ch:(h + 1) * out_ch]
        a_s = jnp.sum(xh * atts_ref[h:h + 1, :], axis=1, keepdims=True)
        a_d = lax.dot_general(attd_ref[h:h + 1, :], xhd,
                              (((1,), (1,)), ((), ())),
                              preferred_element_type=jnp.float32,
                              precision=lax.Precision.HIGHEST)
        logit = a_s + a_d
        logit = jnp.where(logit >= 0.0, logit, 0.2 * logit)
        e = jnp.exp(logit)
        w = jnp.where(msk, e, 0.0) + jnp.where(diag, e, 0.0)
        denom = jnp.sum(w, axis=0, keepdims=True) + 1e-16
        w = w * (1.0 / denom)
        w_hi = w.astype(jnp.bfloat16).astype(jnp.float32)
        w_lo = w - w_hi
        x_hi = xh.astype(jnp.bfloat16).astype(jnp.float32)
        x_lo = xh - x_hi
        dims = (((0,), (0,)), ((), ()))
        o = (lax.dot_general(w_hi, x_hi, dims,
                             preferred_element_type=jnp.float32)
             + lax.dot_general(w_hi, x_lo, dims,
                               preferred_element_type=jnp.float32)
             + lax.dot_general(w_lo, x_hi, dims,
                               preferred_element_type=jnp.float32))
        outs.append(o)

    out = outs[0] if heads == 1 else jnp.concatenate(outs, axis=1)
    out = out + bias_ref[0:1, :]
    if do_elu:
        out = jnp.where(out > 0.0, out, jnp.exp(out) - 1.0)
    out_ref[...] = out


def _gat_layer(agg, x, W, att_src, att_dst, mode, allow, thresh, bias,
               heads, out_ch, do_elu):
    body = functools.partial(_gat_body, heads=heads, out_ch=out_ch,
                             do_elu=do_elu)
    feat = heads * out_ch
    in_ch = x.shape[1]
    return pl.pallas_call(
        body,
        grid=(N // BD,),
        in_specs=[
            pl.BlockSpec((N, BD), lambda j: (0, j)),
            pl.BlockSpec((N, in_ch), lambda j: (0, 0)),
            pl.BlockSpec((BD, in_ch), lambda j: (j, 0)),
            pl.BlockSpec((in_ch, feat), lambda j: (0, 0)),
            pl.BlockSpec((heads, out_ch), lambda j: (0, 0)),
            pl.BlockSpec((heads, out_ch), lambda j: (0, 0)),
            pl.BlockSpec((N, 1), lambda j: (0, 0)),
            pl.BlockSpec((1, N), lambda j: (0, 0)),
            pl.BlockSpec(memory_space=pltpu.SMEM),
            pl.BlockSpec((1, feat), lambda j: (0, 0)),
        ],
        out_specs=pl.BlockSpec((BD, feat), lambda j: (j, 0)),
        out_shape=jax.ShapeDtypeStruct((N, feat), jnp.float32),
    )(agg, x, x, W, att_src.reshape(heads, out_ch),
      att_dst.reshape(heads, out_ch), mode, allow, thresh.reshape(1, 1),
      bias.reshape(1, feat))


def _bisect_step(lo, hi, cnts, pivots, target):
    sel = jnp.sum((cnts >= target).astype(jnp.int32))
    piv_ext = jnp.concatenate([pivots, hi[None]])
    lo = piv_ext[sel - 1]
    hi = piv_ext[sel]
    return lo, hi, sel


def _pivots(lo, hi):
    return lo + (hi - lo) / NPIV * jnp.arange(NPIV, dtype=jnp.float32)


def kernel(x, attn_tensor, agg_w, agg_b, W1, att_src1, att_dst1, b1,
           W2, att_src2, att_dst2, b2):
    target = jnp.float32(NUM_EDGES)
    lo = jnp.sum(jnp.minimum(agg_w, 0.0)) + agg_b - 1e-3
    hi = jnp.sum(jnp.maximum(agg_w, 0.0)) + agg_b + 1e-3

    piv1 = _pivots(lo, hi)
    agg, cnt1 = _agg_mat(attn_tensor, agg_w, agg_b, piv1)
    lo, hi, _ = _bisect_step(lo, hi, cnt1[0, :NPIV], piv1, target)

    hist = _sc_hist(agg.reshape(-1), lo, hi)
    sfx = jnp.cumsum(hist[::-1])[::-1]
    b = jnp.sum((sfx >= target).astype(jnp.int32)) - 1
    thresh = lo + b.astype(jnp.float32) * (hi - lo) / NBINS_SC
    rcnt = _row_counts(agg, thresh)[:, 0]

    excl = jnp.cumsum(rcnt) - rcnt
    limit = jnp.clip(target - excl, 0.0, rcnt)
    mode = jnp.where(limit >= rcnt, 1.0,
                     jnp.where(limit > 0.0, 2.0, 0.0))
    is_part = mode == 2.0
    rstar = jnp.argmax(is_part)
    rowvals = lax.dynamic_slice(agg, (rstar, 0), (1, N))[0]
    inrow = (rowvals >= thresh).astype(jnp.float32)
    pfx = jnp.cumsum(inrow) - inrow
    lim_r = limit[rstar]
    allow = jnp.where(jnp.any(is_part), (pfx < lim_r).astype(jnp.float32),
                      jnp.ones((N,), jnp.float32))
    mode = mode.reshape(N, 1)
    allow = allow.reshape(1, N)

    h1 = _gat_layer(agg, x, W1, att_src1, att_dst1, mode, allow, thresh, b1,
                    HEADS, HID, do_elu=True)

    out = _gat_layer(agg, h1, W2, att_src2, att_dst2, mode, allow, thresh, b2,
                     1, OUT_CH, do_elu=False)
    return out


# speedup vs baseline: 1.1516x; 1.0134x over previous
import functools
import jax
import jax.numpy as jnp
from jax import lax
from jax.experimental import pallas as pl
from jax.experimental.pallas import tpu as pltpu
from jax.experimental.pallas import tpu_sc as plsc

N = 2048
IN_CH = 128
HID = 64
HEADS = 4
OUT_CH = 128

M_TOTAL = N * N
NUM_EDGES = M_TOTAL - 1 - (9 * (M_TOTAL - 1)) // 10

ROWS_A = 64
ROWS_C = 256
BD = 512
NPIV = 16


def _count_lanes(blk, piv_ref):
    lane = lax.broadcasted_iota(jnp.int32, (1, 128), 1)
    acc = jnp.zeros((1, 128), jnp.float32)
    for p in range(NPIV):
        cnt = jnp.sum((blk >= piv_ref[0, p]).astype(jnp.float32))
        acc = acc + jnp.where(lane == p, cnt, 0.0)
    return acc


def _agg_body(attn_ref, w_ref, b_ref, piv_ref, out_ref, cnt_ref):
    def rnd(v):
        return v.astype(jnp.bfloat16).astype(jnp.float32)

    acc = rnd(w_ref[0, 0]) * rnd(attn_ref[0, 0])
    for c in range(1, 12):
        acc = acc + rnd(w_ref[0, c]) * rnd(attn_ref[0, c])
    agg = acc + b_ref[0, 0]
    out_ref[...] = agg

    @pl.when(pl.program_id(0) == 0)
    def _init():
        cnt_ref[...] = jnp.zeros_like(cnt_ref)

    cnt_ref[...] += _count_lanes(agg, piv_ref)


def _agg_mat(attn, agg_w, agg_b, pivots):
    piv = jnp.full((1, 128), jnp.inf, jnp.float32).at[0, :NPIV].set(pivots)
    return pl.pallas_call(
        _agg_body,
        grid=(N // ROWS_A,),
        in_specs=[
            pl.BlockSpec((1, 12, ROWS_A, N), lambda i: (0, 0, i, 0)),
            pl.BlockSpec(memory_space=pltpu.SMEM),
            pl.BlockSpec(memory_space=pltpu.SMEM),
            pl.BlockSpec(memory_space=pltpu.SMEM),
        ],
        out_specs=[
            pl.BlockSpec((ROWS_A, N), lambda i: (i, 0)),
            pl.BlockSpec((1, 128), lambda i: (0, 0)),
        ],
        out_shape=[
            jax.ShapeDtypeStruct((N, N), jnp.float32),
            jax.ShapeDtypeStruct((1, 128), jnp.float32),
        ],
    )(attn, agg_w.reshape(1, 12), agg_b.reshape(1, 1), piv)


def _count_body(agg_ref, piv_ref, out_ref):
    @pl.when(pl.program_id(0) == 0)
    def _init():
        out_ref[...] = jnp.zeros_like(out_ref)

    out_ref[...] += _count_lanes(agg_ref[...], piv_ref)


def _count_ge(agg, pivots):
    piv = jnp.full((1, 128), jnp.inf, jnp.float32).at[0, :NPIV].set(pivots)
    out = pl.pallas_call(
        _count_body,
        grid=(N // ROWS_C,),
        in_specs=[
            pl.BlockSpec((ROWS_C, N), lambda i: (i, 0)),
            pl.BlockSpec(memory_space=pltpu.SMEM),
        ],
        out_specs=pl.BlockSpec((1, 128), lambda i: (0, 0)),
        out_shape=jax.ShapeDtypeStruct((1, 128), jnp.float32),
    )(agg, piv)
    return out[0, :NPIV]


NBINS_SC = 4096
SC_NW = 32
SC_CHUNK = 32768
SC_ELEMS = M_TOTAL // SC_NW


def _sc_hist_body(agg_ref, sc_ref, c_ref, zero_ref, out_ref,
                  chunk, hist, scvec, cvec):
    wid = lax.axis_index("s") * 2 + lax.axis_index("c")
    base = wid * SC_ELEMS
    pltpu.sync_copy(sc_ref, scvec)
    pltpu.sync_copy(c_ref, cvec)
    pltpu.sync_copy(zero_ref, hist)
    sc_v = scvec[...]
    c_v = cvec[...]
    lane2 = lax.iota(jnp.int32, 16) - 16 * 131072
    ones = jnp.full((16,), 1.0, jnp.float32)
    for c in range(SC_ELEMS // SC_CHUNK):
        pltpu.sync_copy(agg_ref.at[pl.ds(base + c * SC_CHUNK, SC_CHUNK)],
                        chunk)

        def body(i, carry):
            for u in range(8):
                v = chunk[pl.ds(i * 128 + u * 16, 16)]
                t = v * sc_v + c_v
                t = jnp.minimum(jnp.maximum(t, 131072.0),
                                131072.0 + (NBINS_SC - 1))
                idx = t.astype(jnp.int32) * 16 + lane2
                plsc.addupdate_scatter(hist, [idx], ones)
            return carry

        lax.fori_loop(0, SC_CHUNK // 128, body, 0)
    pltpu.sync_copy(hist, out_ref.at[wid])


def _sc_hist(agg1d, lo, hi):
    scale = jnp.float32(NBINS_SC) / (hi - lo)
    sc16 = jnp.full((16,), scale, jnp.float32)
    c16 = jnp.full((16,), 131072.0 - lo * scale, jnp.float32)
    zeros = jnp.zeros((NBINS_SC * 16,), jnp.float32)
    mesh = plsc.VectorSubcoreMesh(core_axis_name="c", subcore_axis_name="s")
    f = pl.kernel(
        _sc_hist_body,
        out_type=jax.ShapeDtypeStruct((SC_NW, NBINS_SC * 16), jnp.float32),
        mesh=mesh,
        compiler_params=pltpu.CompilerParams(needs_layout_passes=False),
        scratch_types=[
            pltpu.VMEM((SC_CHUNK,), jnp.float32),
            pltpu.VMEM((NBINS_SC * 16,), jnp.float32),
            pltpu.VMEM((16,), jnp.float32),
            pltpu.VMEM((16,), jnp.float32),
        ],
    )
    out = f(agg1d, sc16, c16, zeros)
    return jnp.sum(out.reshape(SC_NW, NBINS_SC, 16), axis=(0, 2))


def _row_count_body(agg_ref, t_ref, out_ref):
    msk = (agg_ref[...] >= t_ref[0, 0]).astype(jnp.float32)
    out_ref[...] = jnp.sum(msk, axis=1, keepdims=True)


def _row_counts(agg, thresh):
    return pl.pallas_call(
        _row_count_body,
        grid=(N // ROWS_C,),
        in_specs=[
            pl.BlockSpec((ROWS_C, N), lambda i: (i, 0)),
            pl.BlockSpec(memory_space=pltpu.SMEM),
        ],
        out_specs=pl.BlockSpec((ROWS_C, 1), lambda i: (i, 0)),
        out_shape=jax.ShapeDtypeStruct((N, 1), jnp.float32),
    )(agg, thresh.reshape(1, 1))


def _gat_body(agg_ref, x_ref, xd_ref, w_ref, atts_ref, attd_ref, mode_ref,
              allow_ref, t_ref, bias_ref, out_ref, *, heads, out_ch, do_elu):
    j = pl.program_id(0)
    dst0 = j * BD
    agg = agg_ref[...]
    mode = mode_ref[...]
    allow = allow_ref[0:1, pl.ds(dst0, BD)]
    msk = (agg >= t_ref[0, 0]) & (
        (mode == 1.0) | ((mode == 2.0) & (allow == 1.0)))

    rows = lax.broadcasted_iota(jnp.int32, (N, BD), 0)
    cols = lax.broadcasted_iota(jnp.int32, (N, BD), 1)
    diag = rows == (cols + dst0)

    xl = jnp.dot(x_ref[...], w_ref[...], preferred_element_type=jnp.float32)
    xld = jnp.dot(xd_ref[...], w_ref[...], preferred_element_type=jnp.float32)

    outs = []
    for h in range(heads):
        xh = xl[:, h * out_ch:(h + 1) * out_ch]
        xhd = xld[:, h * out_ch:(h + 1) * out_ch]
        a_s = jnp.sum(xh * atts_ref[h:h + 1, :], axis=1, keepdims=True)
        a_d = lax.dot_general(attd_ref[h:h + 1, :], xhd,
                              (((1,), (1,)), ((), ())),
                              preferred_element_type=jnp.float32,
                              precision=lax.Precision.HIGHEST)
        logit = a_s + a_d
        logit = jnp.where(logit >= 0.0, logit, 0.2 * logit)
        e = jnp.exp(logit)
        w = jnp.where(msk, e, 0.0) + jnp.where(diag, e, 0.0)
        denom = jnp.sum(w, axis=0, keepdims=True) + 1e-16
        w = w * (1.0 / denom)
        w_hi = w.astype(jnp.bfloat16).astype(jnp.float32)
        w_lo = w - w_hi
        x_hi = xh.astype(jnp.bfloat16).astype(jnp.float32)
        x_lo = xh - x_hi
        dims = (((0,), (0,)), ((), ()))
        o = (lax.dot_general(w_hi, x_hi, dims,
                             preferred_element_type=jnp.float32)
             + lax.dot_general(w_hi, x_lo, dims,
                               preferred_element_type=jnp.float32)
             + lax.dot_general(w_lo, x_hi, dims,
                               preferred_element_type=jnp.float32))
        outs.append(o)

    out = outs[0] if heads == 1 else jnp.concatenate(outs, axis=1)
    out = out + bias_ref[0:1, :]
    if do_elu:
        out = jnp.where(out > 0.0, out, jnp.exp(out) - 1.0)
    out_ref[...] = out


def _gat_layer(agg, x, W, att_src, att_dst, mode, allow, thresh, bias,
               heads, out_ch, do_elu):
    body = functools.partial(_gat_body, heads=heads, out_ch=out_ch,
                             do_elu=do_elu)
    feat = heads * out_ch
    in_ch = x.shape[1]
    return pl.pallas_call(
        body,
        grid=(N // BD,),
        in_specs=[
            pl.BlockSpec((N, BD), lambda j: (0, j)),
            pl.BlockSpec((N, in_ch), lambda j: (0, 0)),
            pl.BlockSpec((BD, in_ch), lambda j: (j, 0)),
            pl.BlockSpec((in_ch, feat), lambda j: (0, 0)),
            pl.BlockSpec((heads, out_ch), lambda j: (0, 0)),
            pl.BlockSpec((heads, out_ch), lambda j: (0, 0)),
            pl.BlockSpec((N, 1), lambda j: (0, 0)),
            pl.BlockSpec((1, N), lambda j: (0, 0)),
            pl.BlockSpec(memory_space=pltpu.SMEM),
            pl.BlockSpec((1, feat), lambda j: (0, 0)),
        ],
        out_specs=pl.BlockSpec((BD, feat), lambda j: (j, 0)),
        out_shape=jax.ShapeDtypeStruct((N, feat), jnp.float32),
    )(agg, x, x, W, att_src.reshape(heads, out_ch),
      att_dst.reshape(heads, out_ch), mode, allow, thresh.reshape(1, 1),
      bias.reshape(1, feat))


def _bisect_step(lo, hi, cnts, pivots, target):
    sel = jnp.sum((cnts >= target).astype(jnp.int32))
    piv_ext = jnp.concatenate([pivots, hi[None]])
    lo = piv_ext[sel - 1]
    hi = piv_ext[sel]
    return lo, hi, sel


def _pivots(lo, hi):
    return lo + (hi - lo) / NPIV * jnp.arange(NPIV, dtype=jnp.float32)


def kernel(x, attn_tensor, agg_w, agg_b, W1, att_src1, att_dst1, b1,
           W2, att_src2, att_dst2, b2):
    target = jnp.float32(NUM_EDGES)
    lo = jnp.sum(jnp.minimum(agg_w, 0.0)) + agg_b - 1e-3
    hi = jnp.sum(jnp.maximum(agg_w, 0.0)) + agg_b + 1e-3

    piv1 = _pivots(lo, hi)
    agg, cnt1 = _agg_mat(attn_tensor, agg_w, agg_b, piv1)
    lo, hi, _ = _bisect_step(lo, hi, cnt1[0, :NPIV], piv1, target)

    hist = _sc_hist(agg.reshape(-1), lo, hi)
    sfx = jnp.cumsum(hist[::-1])[::-1]
    b = jnp.sum((sfx >= target).astype(jnp.int32)) - 1
    thresh = lo + b.astype(jnp.float32) * (hi - lo) / NBINS_SC
    rcnt = _row_counts(agg, thresh)[:, 0]

    excl = jnp.cumsum(rcnt) - rcnt
    limit = jnp.clip(target - excl, 0.0, rcnt)
    mode = jnp.where(limit >= rcnt, 1.0,
                     jnp.where(limit > 0.0, 2.0, 0.0))
    is_part = mode == 2.0
    rstar = jnp.argmax(is_part)
    rowvals = lax.dynamic_slice(agg, (rstar, 0), (1, N))[0]
    inrow = (rowvals >= thresh).astype(jnp.float32)
    pfx = jnp.cumsum(inrow) - inrow
    lim_r = limit[rstar]
    allow = jnp.where(jnp.any(is_part), (pfx < lim_r).astype(jnp.float32),
                      jnp.ones((N,), jnp.float32))
    mode = mode.reshape(N, 1)
    allow = allow.reshape(1, N)

    h1 = _gat_layer(agg, x, W1, att_src1, att_dst1, mode, allow, thresh, b1,
                    HEADS, HID, do_elu=True)

    out = _gat_layer(agg, h1, W2, att_src2, att_dst2, mode, allow, thresh, b2,
                     1, OUT_CH, do_elu=False)
    return out
